# baseline (speedup 1.0000x reference)
import functools
import math

import jax
import jax.numpy as jnp
from jax import lax
from jax.experimental import pallas as pl
from jax.experimental.pallas import tpu as pltpu
from jax.experimental.pallas import tpu_sc as plsc

_SCALE = math.sqrt(32.0)

_NC = 2
_NS = 16
_NW = _NC * _NS

_CHUNK = 128


def _embed_kernel(x_hbm, table_hbm, out_hbm, idx_v, rows_v, sem):
    nchunk = idx_v.shape[0]
    ch, d = rows_v.shape
    wid = lax.axis_index("s") * _NC + lax.axis_index("c")
    pltpu.sync_copy(x_hbm.at[wid], idx_v)
    base = wid * nchunk * ch

    def chunk_body(t, carry):
        pltpu.async_copy(table_hbm.at[idx_v.at[t]], rows_v, sem).wait()

        def scale_body(i, c):
            rows_v[i, pl.ds(0, 16)] = rows_v[i, pl.ds(0, 16)] * _SCALE
            rows_v[i, pl.ds(16, 16)] = rows_v[i, pl.ds(16, 16)] * _SCALE
            return c

        lax.fori_loop(0, ch, scale_body, 0, unroll=4)
        pltpu.sync_copy(rows_v, out_hbm.at[pl.ds(base + t * ch, ch)])
        return carry

    lax.fori_loop(0, nchunk, chunk_body, 0)


def kernel(x, table):
    b0, s = x.shape
    v, d = table.shape
    b = b0 * s
    assert b % (_NW * _CHUNK) == 0
    nchunk = b // (_NW * _CHUNK)
    xf = x.reshape(_NW, nchunk, _CHUNK).astype(jnp.int32)

    mesh = plsc.VectorSubcoreMesh(
        core_axis_name="c", subcore_axis_name="s", num_cores=_NC,
        num_subcores=_NS)
    run = functools.partial(
        pl.kernel,
        out_type=jax.ShapeDtypeStruct((b, d), jnp.float32),
        mesh=mesh,
        compiler_params=pltpu.CompilerParams(use_tc_tiling_on_sc=False),
        scratch_types=[
            pltpu.VMEM((nchunk, _CHUNK), jnp.int32),
            pltpu.VMEM((_CHUNK, d), jnp.float32),
            pltpu.SemaphoreType.DMA,
        ],
    )(_embed_kernel)
    out = run(xf, table)
    return out.reshape(b0, s, d)
